# baseline (speedup 1.0000x reference)
import functools

import jax
import jax.numpy as jnp
from jax import lax
from jax.experimental import pallas as pl
from jax.experimental.pallas import tpu as pltpu
from jax.experimental.pallas import tpu_sc as plsc

N_NODES = 10000
NNZ = 320000
N_GRAPHS = 64
D = 128
ROWS = 10240
TRASH = 10000
NW = 32
CHUNK = 128
K = 80
NNZP = NW * K * CHUNK
RB = ROWS // 16
BLK = 1024
GRID = ROWS // BLK


@functools.cache
def _get_sc_pass():
    mesh = plsc.VectorSubcoreMesh(core_axis_name="c", subcore_axis_name="s")

    @functools.partial(
        pl.kernel,
        out_type=jax.ShapeDtypeStruct((2, ROWS, D), jnp.float32),
        mesh=mesh,
        scratch_types=[
            pltpu.VMEM((K // 2, CHUNK), jnp.int32),
            pltpu.VMEM((K // 2, CHUNK), jnp.int32),
            pltpu.VMEM((CHUNK, D), jnp.float32),
            pltpu.VMEM((CHUNK, D), jnp.float32),
            pltpu.VMEM_SHARED((ROWS, D), jnp.float32),
            pltpu.SemaphoreType.DMA,
            pltpu.SemaphoreType.DMA,
            pltpu.SemaphoreType.DMA,
            pltpu.SemaphoreType.DMA,
        ],
    )
    def sc_pass(src, gidx, sidx, zrows, out, gv, sv, buf_a, buf_b, acc,
                gsem_a, gsem_b, ssem_a, ssem_b):
        c = lax.axis_index("c")
        s = lax.axis_index("s")
        w = s * 2 + c
        pltpu.sync_copy(zrows.at[pl.ds(s * RB, RB)], acc.at[pl.ds(s * RB, RB)])
        plsc.subcore_barrier()

        for half in range(2):
            row0 = w * K + half * (K // 2)
            pltpu.sync_copy(gidx.at[pl.ds(row0, K // 2)], gv)
            pltpu.sync_copy(sidx.at[pl.ds(row0, K // 2)], sv)
            pltpu.async_copy(src.at[gv.at[0]], buf_a, gsem_a).wait()

            def step(j, carry):
                db = pltpu.async_copy(src.at[gv.at[2 * j + 1]], buf_b, gsem_b)
                sa = pltpu.async_copy(buf_a, acc.at[sv.at[2 * j]], ssem_a,
                                      add=True)
                db.wait()
                sb = pltpu.async_copy(buf_b, acc.at[sv.at[2 * j + 1]], ssem_b,
                                      add=True)
                sa.wait()

                @pl.when(j < K // 4 - 1)
                def _prefetch():
                    pltpu.async_copy(src.at[gv.at[2 * j + 2]], buf_a,
                                     gsem_a).wait()

                sb.wait()
                return carry

            lax.fori_loop(0, K // 4, step, 0)
        plsc.subcore_barrier()
        pltpu.sync_copy(acc.at[pl.ds(s * RB, RB)],
                        out.at[c, pl.ds(s * RB, RB)])

    return sc_pass


@functools.cache
def _get_sc_counts():
    mesh = plsc.VectorSubcoreMesh(core_axis_name="c", subcore_axis_name="s")

    @functools.partial(
        pl.kernel,
        out_type=(jax.ShapeDtypeStruct((NW, ROWS), jnp.float32),
                  jax.ShapeDtypeStruct((NW, ROWS), jnp.float32)),
        mesh=mesh,
        scratch_types=[
            pltpu.VMEM((K, CHUNK), jnp.int32),
            pltpu.VMEM((K, CHUNK), jnp.int32),
            pltpu.VMEM((ROWS,), jnp.float32),
            pltpu.VMEM((ROWS,), jnp.float32),
        ],
        compiler_params=pltpu.CompilerParams(needs_layout_passes=False),
    )
    def sc_counts(gidx, sidx, out_d, out_b, gv, sv, cd, cb):
        c = lax.axis_index("c")
        s = lax.axis_index("s")
        w = s * 2 + c
        pltpu.sync_copy(gidx.at[pl.ds(w * K, K)], gv)
        pltpu.sync_copy(sidx.at[pl.ds(w * K, K)], sv)
        zero16 = jnp.zeros((16,), jnp.float32)

        def zstep(i, carry):
            off = pl.multiple_of(i * 16, 16)
            cd[pl.ds(off, 16)] = zero16
            cb[pl.ds(off, 16)] = zero16
            return carry

        lax.fori_loop(0, ROWS // 16, zstep, 0)
        one16 = jnp.full((16,), 1.0, jnp.float32)

        def cstep(j, carry):
            for col in range(CHUNK // 16):
                gi = gv[j, pl.ds(col * 16, 16)]
                si = sv[j, pl.ds(col * 16, 16)]
                plsc.addupdate_scatter(cd, [gi], one16)
                plsc.addupdate_scatter(cb, [si], one16)
            return carry

        lax.fori_loop(0, K, cstep, 0)
        pltpu.sync_copy(cd, out_d.at[w])
        pltpu.sync_copy(cb, out_b.at[w])

    return sc_counts


def _mm_inv(xp, w, cnt_d, cnt_b):
    def body(x_ref, w_ref, d_ref, b_ref, o_ref, od_ref, ob_ref):
        o_ref[...] = jnp.dot(x_ref[...], w_ref[...],
                             preferred_element_type=jnp.float32)
        eye = (lax.broadcasted_iota(jnp.int32, (BLK, BLK), 0) ==
               lax.broadcasted_iota(jnp.int32, (BLK, BLK), 1)
               ).astype(jnp.float32)
        for cref, oref in ((d_ref, od_ref), (b_ref, ob_ref)):
            st = jnp.sum(cref[...], axis=0, keepdims=True)
            inv = jnp.where(st > 0, 1.0 / st, 0.0)
            oref[...] = lax.dot_general(
                eye, inv, (((1,), (1,)), ((), ())),
                preferred_element_type=jnp.float32)
    return pl.pallas_call(
        body, grid=(GRID,),
        in_specs=[pl.BlockSpec((BLK, D), lambda i: (i, 0)),
                  pl.BlockSpec((D, D), lambda i: (0, 0)),
                  pl.BlockSpec((NW, BLK), lambda i: (0, i)),
                  pl.BlockSpec((NW, BLK), lambda i: (0, i))],
        out_specs=[pl.BlockSpec((BLK, D), lambda i: (i, 0)),
                   pl.BlockSpec((BLK, 1), lambda i: (i, 0)),
                   pl.BlockSpec((BLK, 1), lambda i: (i, 0))],
        out_shape=[jax.ShapeDtypeStruct((ROWS, D), jnp.float32),
                   jax.ShapeDtypeStruct((ROWS, 1), jnp.float32),
                   jax.ShapeDtypeStruct((ROWS, 1), jnp.float32)],
    )(xp, w, cnt_d, cnt_b)


def _comb(p0, p1, inv):
    def body(a_ref, b_ref, i_ref, o_ref):
        o_ref[...] = (a_ref[...] + b_ref[...]) * i_ref[...]
    return pl.pallas_call(
        body, grid=(GRID,),
        in_specs=[pl.BlockSpec((BLK, D), lambda i: (i, 0)),
                  pl.BlockSpec((BLK, D), lambda i: (i, 0)),
                  pl.BlockSpec((BLK, 1), lambda i: (i, 0))],
        out_specs=pl.BlockSpec((BLK, D), lambda i: (i, 0)),
        out_shape=jax.ShapeDtypeStruct((ROWS, D), jnp.float32),
    )(p0, p1, inv)


def _relu_mm(n0, n1, dinv, bias, w):
    def body(a_ref, b_ref, i_ref, bias_ref, w_ref, ox_ref, ow_ref):
        xv = jnp.maximum(
            (a_ref[...] + b_ref[...]) * i_ref[...] + bias_ref[...], 0.0)
        ox_ref[...] = xv
        ow_ref[...] = jnp.dot(xv, w_ref[...],
                              preferred_element_type=jnp.float32)
    return pl.pallas_call(
        body, grid=(GRID,),
        in_specs=[pl.BlockSpec((BLK, D), lambda i: (i, 0)),
                  pl.BlockSpec((BLK, D), lambda i: (i, 0)),
                  pl.BlockSpec((BLK, 1), lambda i: (i, 0)),
                  pl.BlockSpec((1, D), lambda i: (0, 0)),
                  pl.BlockSpec((D, D), lambda i: (0, 0))],
        out_specs=[pl.BlockSpec((BLK, D), lambda i: (i, 0)),
                   pl.BlockSpec((BLK, D), lambda i: (i, 0))],
        out_shape=[jax.ShapeDtypeStruct((ROWS, D), jnp.float32),
                   jax.ShapeDtypeStruct((ROWS, D), jnp.float32)],
    )(n0, n1, dinv, bias, w)


def _final(n0, n1, dinv, bias, x1, bt, wlin, blin):
    def body(a_ref, b_ref, i_ref, bias_ref, x1_ref, bt_ref, wl_ref, bl_ref,
             o_ref, s_acc, c_acc, m_acc):
        i = pl.program_id(0)

        @pl.when(i == 0)
        def _init():
            s_acc[...] = jnp.zeros_like(s_acc)
            c_acc[...] = jnp.zeros_like(c_acc)
            m_acc[...] = jnp.full_like(m_acc, -jnp.inf)

        x2 = jnp.maximum(
            (a_ref[...] + b_ref[...]) * i_ref[...] + bias_ref[...], 0.0)
        h = x1_ref[...] + x2
        bcol = bt_ref[...]
        gids = lax.broadcasted_iota(jnp.int32, (BLK, N_GRAPHS), 1)
        oh = (bcol == gids).astype(jnp.float32)
        s_acc[...] += lax.dot_general(oh, h, (((0,), (0,)), ((), ())),
                                      preferred_element_type=jnp.float32)
        c_acc[...] += jnp.sum(oh, axis=0, keepdims=True)
        lo = jnp.min(bcol)
        hi = jnp.max(bcol)
        for g in range(N_GRAPHS):
            @pl.when((g >= lo) & (g <= hi))
            def _upd(g=g):
                hm = jnp.where(bcol == g, h, -jnp.inf)
                m_acc[g:g + 1, :] = jnp.maximum(
                    m_acc[g:g + 1, :], jnp.max(hm, axis=0, keepdims=True))

        @pl.when(i == GRID - 1)
        def _fin():
            s = s_acc[...]
            eye = (lax.broadcasted_iota(jnp.int32, (N_GRAPHS, N_GRAPHS), 0) ==
                   lax.broadcasted_iota(jnp.int32, (N_GRAPHS, N_GRAPHS), 1)
                   ).astype(jnp.float32)
            cnt = lax.dot_general(eye, c_acc[...], (((1,), (1,)), ((), ())),
                                  preferred_element_type=jnp.float32)
            mean = s / jnp.maximum(cnt, 1.0)
            pooled = jnp.concatenate([mean, m_acc[...], s], axis=1)
            o_ref[...] = jnp.dot(pooled, wl_ref[...],
                                 preferred_element_type=jnp.float32) + bl_ref[...]

    return pl.pallas_call(
        body, grid=(GRID,),
        in_specs=[pl.BlockSpec((BLK, D), lambda i: (i, 0)),
                  pl.BlockSpec((BLK, D), lambda i: (i, 0)),
                  pl.BlockSpec((BLK, 1), lambda i: (i, 0)),
                  pl.BlockSpec((1, D), lambda i: (0, 0)),
                  pl.BlockSpec((BLK, D), lambda i: (i, 0)),
                  pl.BlockSpec((BLK, 1), lambda i: (i, 0)),
                  pl.BlockSpec((3 * D, D), lambda i: (0, 0)),
                  pl.BlockSpec((1, D), lambda i: (0, 0))],
        out_specs=pl.BlockSpec((N_GRAPHS, D), lambda i: (0, 0)),
        out_shape=jax.ShapeDtypeStruct((N_GRAPHS, D), jnp.float32),
        scratch_shapes=[pltpu.VMEM((N_GRAPHS, D), jnp.float32),
                        pltpu.VMEM((1, N_GRAPHS), jnp.float32),
                        pltpu.VMEM((N_GRAPHS, D), jnp.float32)],
    )(n0, n1, dinv, bias, x1, bt, wlin, blin)


def kernel(x, hyperedge_index, batch, W1, b1, W2, b2, Wlin, blin):
    node_idx = hyperedge_index[0]
    edge_idx = hyperedge_index[1]
    pad = TRASH + (jnp.arange(NNZP - NNZ, dtype=jnp.int32) % (ROWS - TRASH))
    nidx = jnp.concatenate([node_idx, pad]).reshape(NW * K, CHUNK)
    eidx = jnp.concatenate([edge_idx, pad]).reshape(NW * K, CHUNK)
    xp = jnp.pad(x, ((0, ROWS - N_NODES), (0, 0)))
    bt = jnp.concatenate(
        [batch, jnp.full((ROWS - N_NODES,), N_GRAPHS, jnp.int32)]
    ).reshape(ROWS, 1)
    zrows = jnp.zeros((ROWS, D), jnp.float32)
    b1r = b1.reshape(1, D)
    b2r = b2.reshape(1, D)
    blr = blin.reshape(1, D)

    sc_counts = _get_sc_counts()
    sc_pass = _get_sc_pass()

    cnt_d, cnt_b = sc_counts(nidx, eidx)
    xw1, dinv, binv = _mm_inv(xp, W1, cnt_d, cnt_b)
    ep = sc_pass(xw1, nidx, eidx, zrows)
    ec = _comb(ep[0], ep[1], binv)
    np_ = sc_pass(ec, eidx, nidx, zrows)
    x1, xw2 = _relu_mm(np_[0], np_[1], dinv, b1r, W2)

    ep2 = sc_pass(xw2, nidx, eidx, zrows)
    ec2 = _comb(ep2[0], ep2[1], binv)
    np2 = sc_pass(ec2, eidx, nidx, zrows)

    return _final(np2[0], np2[1], dinv, b2r, x1, bt, Wlin, blr)


# speedup vs baseline: 1.0337x; 1.0337x over previous
import functools

import jax
import jax.numpy as jnp
from jax import lax
from jax.experimental import pallas as pl
from jax.experimental.pallas import tpu as pltpu
from jax.experimental.pallas import tpu_sc as plsc

N_NODES = 10000
NNZ = 320000
N_GRAPHS = 64
D = 128
ROWS = 10240
TRASH = 10000
NW = 32
CHUNK = 128
K = 80
NNZP = NW * K * CHUNK
RB = ROWS // 16
BLK = 1024
GRID = ROWS // BLK


@functools.cache
def _get_sc_pass():
    mesh = plsc.VectorSubcoreMesh(core_axis_name="c", subcore_axis_name="s")

    @functools.partial(
        pl.kernel,
        out_type=jax.ShapeDtypeStruct((2, ROWS, D), jnp.float32),
        mesh=mesh,
        scratch_types=[
            pltpu.VMEM((K // 2, CHUNK), jnp.int32),
            pltpu.VMEM((K // 2, CHUNK), jnp.int32),
            pltpu.VMEM((CHUNK, D), jnp.float32),
            pltpu.VMEM((CHUNK, D), jnp.float32),
            pltpu.VMEM_SHARED((ROWS, D), jnp.float32),
            pltpu.SemaphoreType.DMA,
            pltpu.SemaphoreType.DMA,
            pltpu.SemaphoreType.DMA,
            pltpu.SemaphoreType.DMA,
        ],
    )
    def sc_pass(src, gidx, sidx, zrows, out, gv, sv, buf_a, buf_b, acc,
                gsem_a, gsem_b, ssem_a, ssem_b):
        c = lax.axis_index("c")
        s = lax.axis_index("s")
        w = s * 2 + c
        zdma = pltpu.async_copy(zrows.at[pl.ds(s * RB, RB)],
                                acc.at[pl.ds(s * RB, RB)], ssem_a)

        for half in range(2):
            row0 = w * K + half * (K // 2)
            pltpu.sync_copy(gidx.at[pl.ds(row0, K // 2)], gv)
            pltpu.sync_copy(sidx.at[pl.ds(row0, K // 2)], sv)
            pltpu.async_copy(src.at[gv.at[0]], buf_a, gsem_a).wait()
            if half == 0:
                zdma.wait()
                plsc.subcore_barrier()

            def step(j, carry):
                db = pltpu.async_copy(src.at[gv.at[2 * j + 1]], buf_b, gsem_b)
                sa = pltpu.async_copy(buf_a, acc.at[sv.at[2 * j]], ssem_a,
                                      add=True)
                db.wait()
                sb = pltpu.async_copy(buf_b, acc.at[sv.at[2 * j + 1]], ssem_b,
                                      add=True)
                sa.wait()

                @pl.when(j < K // 4 - 1)
                def _prefetch():
                    pltpu.async_copy(src.at[gv.at[2 * j + 2]], buf_a,
                                     gsem_a).wait()

                sb.wait()
                return carry

            lax.fori_loop(0, K // 4, step, 0)
        plsc.subcore_barrier()
        pltpu.sync_copy(acc.at[pl.ds(s * RB, RB)],
                        out.at[c, pl.ds(s * RB, RB)])

    return sc_pass


@functools.cache
def _get_sc_counts():
    mesh = plsc.VectorSubcoreMesh(core_axis_name="c", subcore_axis_name="s")

    @functools.partial(
        pl.kernel,
        out_type=(jax.ShapeDtypeStruct((NW, ROWS), jnp.float32),
                  jax.ShapeDtypeStruct((NW, ROWS), jnp.float32)),
        mesh=mesh,
        scratch_types=[
            pltpu.VMEM((K, CHUNK), jnp.int32),
            pltpu.VMEM((K, CHUNK), jnp.int32),
            pltpu.VMEM((ROWS,), jnp.float32),
            pltpu.VMEM((ROWS,), jnp.float32),
        ],
        compiler_params=pltpu.CompilerParams(needs_layout_passes=False),
    )
    def sc_counts(gidx, sidx, out_d, out_b, gv, sv, cd, cb):
        c = lax.axis_index("c")
        s = lax.axis_index("s")
        w = s * 2 + c
        pltpu.sync_copy(gidx.at[pl.ds(w * K, K)], gv)
        pltpu.sync_copy(sidx.at[pl.ds(w * K, K)], sv)
        zero16 = jnp.zeros((16,), jnp.float32)

        def zstep(i, carry):
            off = pl.multiple_of(i * 16, 16)
            cd[pl.ds(off, 16)] = zero16
            cb[pl.ds(off, 16)] = zero16
            return carry

        lax.fori_loop(0, ROWS // 16, zstep, 0)
        one16 = jnp.full((16,), 1.0, jnp.float32)

        def cstep(j, carry):
            for col in range(CHUNK // 16):
                gi = gv[j, pl.ds(col * 16, 16)]
                si = sv[j, pl.ds(col * 16, 16)]
                plsc.addupdate_scatter(cd, [gi], one16)
                plsc.addupdate_scatter(cb, [si], one16)
            return carry

        lax.fori_loop(0, K, cstep, 0)
        pltpu.sync_copy(cd, out_d.at[w])
        pltpu.sync_copy(cb, out_b.at[w])

    return sc_counts


def _mm(xp, w):
    def body(x_ref, w_ref, o_ref):
        o_ref[...] = jnp.dot(x_ref[...], w_ref[...],
                             preferred_element_type=jnp.float32)
    return pl.pallas_call(
        body, grid=(GRID,),
        in_specs=[pl.BlockSpec((BLK, D), lambda i: (i, 0)),
                  pl.BlockSpec((D, D), lambda i: (0, 0))],
        out_specs=pl.BlockSpec((BLK, D), lambda i: (i, 0)),
        out_shape=jax.ShapeDtypeStruct((ROWS, D), jnp.float32),
    )(xp, w)


def _inv(cnt_d, cnt_b):
    def body(d_ref, b_ref, od_ref, ob_ref):
        eye = (lax.broadcasted_iota(jnp.int32, (BLK, BLK), 0) ==
               lax.broadcasted_iota(jnp.int32, (BLK, BLK), 1)
               ).astype(jnp.float32)
        for cref, oref in ((d_ref, od_ref), (b_ref, ob_ref)):
            st = jnp.sum(cref[...], axis=0, keepdims=True)
            inv = jnp.where(st > 0, 1.0 / st, 0.0)
            oref[...] = lax.dot_general(
                eye, inv, (((1,), (1,)), ((), ())),
                preferred_element_type=jnp.float32)
    return pl.pallas_call(
        body, grid=(GRID,),
        in_specs=[pl.BlockSpec((NW, BLK), lambda i: (0, i)),
                  pl.BlockSpec((NW, BLK), lambda i: (0, i))],
        out_specs=[pl.BlockSpec((BLK, 1), lambda i: (i, 0)),
                   pl.BlockSpec((BLK, 1), lambda i: (i, 0))],
        out_shape=[jax.ShapeDtypeStruct((ROWS, 1), jnp.float32),
                   jax.ShapeDtypeStruct((ROWS, 1), jnp.float32)],
    )(cnt_d, cnt_b)


def _comb(p0, p1, inv):
    def body(a_ref, b_ref, i_ref, o_ref):
        o_ref[...] = (a_ref[...] + b_ref[...]) * i_ref[...]
    return pl.pallas_call(
        body, grid=(GRID,),
        in_specs=[pl.BlockSpec((BLK, D), lambda i: (i, 0)),
                  pl.BlockSpec((BLK, D), lambda i: (i, 0)),
                  pl.BlockSpec((BLK, 1), lambda i: (i, 0))],
        out_specs=pl.BlockSpec((BLK, D), lambda i: (i, 0)),
        out_shape=jax.ShapeDtypeStruct((ROWS, D), jnp.float32),
    )(p0, p1, inv)


def _relu_mm(n0, n1, dinv, bias, w):
    def body(a_ref, b_ref, i_ref, bias_ref, w_ref, ox_ref, ow_ref):
        xv = jnp.maximum(
            (a_ref[...] + b_ref[...]) * i_ref[...] + bias_ref[...], 0.0)
        ox_ref[...] = xv
        ow_ref[...] = jnp.dot(xv, w_ref[...],
                              preferred_element_type=jnp.float32)
    return pl.pallas_call(
        body, grid=(GRID,),
        in_specs=[pl.BlockSpec((BLK, D), lambda i: (i, 0)),
                  pl.BlockSpec((BLK, D), lambda i: (i, 0)),
                  pl.BlockSpec((BLK, 1), lambda i: (i, 0)),
                  pl.BlockSpec((1, D), lambda i: (0, 0)),
                  pl.BlockSpec((D, D), lambda i: (0, 0))],
        out_specs=[pl.BlockSpec((BLK, D), lambda i: (i, 0)),
                   pl.BlockSpec((BLK, D), lambda i: (i, 0))],
        out_shape=[jax.ShapeDtypeStruct((ROWS, D), jnp.float32),
                   jax.ShapeDtypeStruct((ROWS, D), jnp.float32)],
    )(n0, n1, dinv, bias, w)


def _final(n0, n1, dinv, bias, x1, bt, wlin, blin):
    def body(a_ref, b_ref, i_ref, bias_ref, x1_ref, bt_ref, wl_ref, bl_ref,
             o_ref, s_acc, c_acc, m_acc):
        i = pl.program_id(0)

        @pl.when(i == 0)
        def _init():
            s_acc[...] = jnp.zeros_like(s_acc)
            c_acc[...] = jnp.zeros_like(c_acc)
            m_acc[...] = jnp.full_like(m_acc, -jnp.inf)

        x2 = jnp.maximum(
            (a_ref[...] + b_ref[...]) * i_ref[...] + bias_ref[...], 0.0)
        h = x1_ref[...] + x2
        bcol = bt_ref[...]
        gids = lax.broadcasted_iota(jnp.int32, (BLK, N_GRAPHS), 1)
        oh = (bcol == gids).astype(jnp.float32)
        s_acc[...] += lax.dot_general(oh, h, (((0,), (0,)), ((), ())),
                                      preferred_element_type=jnp.float32)
        c_acc[...] += jnp.sum(oh, axis=0, keepdims=True)
        lo = jnp.min(bcol)
        hi = jnp.max(bcol)
        for g in range(N_GRAPHS):
            @pl.when((g >= lo) & (g <= hi))
            def _upd(g=g):
                hm = jnp.where(bcol == g, h, -jnp.inf)
                m_acc[g:g + 1, :] = jnp.maximum(
                    m_acc[g:g + 1, :], jnp.max(hm, axis=0, keepdims=True))

        @pl.when(i == GRID - 1)
        def _fin():
            s = s_acc[...]
            eye = (lax.broadcasted_iota(jnp.int32, (N_GRAPHS, N_GRAPHS), 0) ==
                   lax.broadcasted_iota(jnp.int32, (N_GRAPHS, N_GRAPHS), 1)
                   ).astype(jnp.float32)
            cnt = lax.dot_general(eye, c_acc[...], (((1,), (1,)), ((), ())),
                                  preferred_element_type=jnp.float32)
            mean = s / jnp.maximum(cnt, 1.0)
            pooled = jnp.concatenate([mean, m_acc[...], s], axis=1)
            o_ref[...] = jnp.dot(pooled, wl_ref[...],
                                 preferred_element_type=jnp.float32) + bl_ref[...]

    return pl.pallas_call(
        body, grid=(GRID,),
        in_specs=[pl.BlockSpec((BLK, D), lambda i: (i, 0)),
                  pl.BlockSpec((BLK, D), lambda i: (i, 0)),
                  pl.BlockSpec((BLK, 1), lambda i: (i, 0)),
                  pl.BlockSpec((1, D), lambda i: (0, 0)),
                  pl.BlockSpec((BLK, D), lambda i: (i, 0)),
                  pl.BlockSpec((BLK, 1), lambda i: (i, 0)),
                  pl.BlockSpec((3 * D, D), lambda i: (0, 0)),
                  pl.BlockSpec((1, D), lambda i: (0, 0))],
        out_specs=pl.BlockSpec((N_GRAPHS, D), lambda i: (0, 0)),
        out_shape=jax.ShapeDtypeStruct((N_GRAPHS, D), jnp.float32),
        scratch_shapes=[pltpu.VMEM((N_GRAPHS, D), jnp.float32),
                        pltpu.VMEM((1, N_GRAPHS), jnp.float32),
                        pltpu.VMEM((N_GRAPHS, D), jnp.float32)],
    )(n0, n1, dinv, bias, x1, bt, wlin, blin)


def kernel(x, hyperedge_index, batch, W1, b1, W2, b2, Wlin, blin):
    node_idx = hyperedge_index[0]
    edge_idx = hyperedge_index[1]
    pad = TRASH + (jnp.arange(NNZP - NNZ, dtype=jnp.int32) % (ROWS - TRASH))
    nidx = jnp.concatenate([node_idx, pad]).reshape(NW * K, CHUNK)
    eidx = jnp.concatenate([edge_idx, pad]).reshape(NW * K, CHUNK)
    xp = jnp.pad(x, ((0, ROWS - N_NODES), (0, 0)))
    bt = jnp.concatenate(
        [batch, jnp.full((ROWS - N_NODES,), N_GRAPHS, jnp.int32)]
    ).reshape(ROWS, 1)
    zrows = jnp.zeros((ROWS, D), jnp.float32)
    b1r = b1.reshape(1, D)
    b2r = b2.reshape(1, D)
    blr = blin.reshape(1, D)

    sc_counts = _get_sc_counts()
    sc_pass = _get_sc_pass()

    cnt_d, cnt_b = sc_counts(nidx, eidx)
    xw1 = _mm(xp, W1)
    dinv, binv = _inv(cnt_d, cnt_b)
    ep = sc_pass(xw1, nidx, eidx, zrows)
    ec = _comb(ep[0], ep[1], binv)
    np_ = sc_pass(ec, eidx, nidx, zrows)
    x1, xw2 = _relu_mm(np_[0], np_[1], dinv, b1r, W2)

    ep2 = sc_pass(xw2, nidx, eidx, zrows)
    ec2 = _comb(ep2[0], ep2[1], binv)
    np2 = sc_pass(ec2, eidx, nidx, zrows)

    return _final(np2[0], np2[1], dinv, b2r, x1, bt, Wlin, blr)
